# baseline (speedup 1.0000x reference)
import functools

import jax
import jax.numpy as jnp
from jax import lax
from jax.experimental import pallas as pl
from jax.experimental.pallas import tpu as pltpu
from jax.experimental.pallas import tpu_sc as plsc

_N_LEVELS = 16
_N_FEAT = 2
_OFFSETS = [0, 4913, 14174, 31750, 67687, 136608, 269259, 543884, 1068172,
            1592460, 2116748, 2641036, 3165324, 3689612, 4213900, 4738188,
            5262476]
_RES = [16, 20, 25, 32, 40, 50, 64, 80, 101, 128, 161, 203, 256, 322, 406, 512]
_P2 = 2654435761
_P3 = 805459861

_B = 131072
_NC, _NS, _L = 2, 16, 16
_NW = _NC * _NS
_BPW = _B // _NW
_C = 128
_CHUNKS = _BPW // _C
_G = _C // _L
_OUT_D = _N_LEVELS * _N_FEAT

_mesh = plsc.VectorSubcoreMesh(core_axis_name="c", subcore_axis_name="s")


@functools.partial(
    pl.kernel,
    mesh=_mesh,
    out_type=jax.ShapeDtypeStruct((_B * _OUT_D,), jnp.float32),
    scratch_types=[
        pltpu.VMEM((3, _L), jnp.float32),
        pltpu.VMEM((3, _L), jnp.float32),
        pltpu.VMEM((3 * _C,), jnp.float32),
        pltpu.VMEM((16, _C), jnp.int32),
        pltpu.VMEM((8 * _C,), jnp.float32),
        pltpu.VMEM((16, _C), jnp.float32),
        pltpu.VMEM((_C * _OUT_D,), jnp.float32),
        pltpu.SemaphoreType.DMA,
    ],
    compiler_params=pltpu.CompilerParams(needs_layout_passes=False),
)
def _encode_sc(xt_hbm, emb_hbm, mn_hbm, inv_hbm, out_hbm,
               mn_v, inv_v, xyz_v, idx_v, w_v, rows_v, out_v, sem):
    wid = lax.axis_index("s") * _NC + lax.axis_index("c")
    base = wid * _BPW
    pltpu.sync_copy(mn_hbm, mn_v)
    pltpu.sync_copy(inv_hbm, inv_v)
    iota = lax.iota(jnp.int32, _L)

    def chunk_body(ci, carry):
        cbase = base + ci * _C
        for a in range(3):
            pltpu.sync_copy(xt_hbm.at[pl.ds(a * _B + cbase, _C)],
                            xyz_v.at[pl.ds(a * _C, _C)])
        for l in range(_N_LEVELS):
            res = _RES[l]
            size = _OFFSETS[l + 1] - _OFFSETS[l]
            off = _OFFSETS[l]
            dense = (res + 1) ** 3 <= size
            r1 = res + 1

            def grp_a(g, cc, res=res, size=size, off=off, dense=dense, r1=r1):
                o = g * _L
                ps, fs = [], []
                for a in range(3):
                    xa = xyz_v[pl.ds(a * _C + o, _L)]
                    xn = jnp.clip((xa - mn_v[a]) * inv_v[a], 0.0, 1.0)
                    scl = xn * jnp.float32(res)
                    p = jnp.minimum(scl.astype(jnp.int32), res - 1)
                    ps.append(p)
                    fs.append(scl - p.astype(jnp.float32))
                px, py, pz = ps
                fx, fy, fz = fs
                wx0 = 1.0 - fx
                wy0 = 1.0 - fy
                wz0 = 1.0 - fz
                wxy = (wx0 * wy0, fx * wy0, wx0 * fy, fx * fy)
                if dense:
                    bidx = px + py * r1 + pz * (r1 * r1) + off
                else:
                    hx0 = px.astype(jnp.uint32)
                    hx1 = hx0 + jnp.uint32(1)
                    hy0 = py.astype(jnp.uint32) * jnp.uint32(_P2)
                    hy1 = hy0 + jnp.uint32(_P2)
                    hz0 = pz.astype(jnp.uint32) * jnp.uint32(_P3)
                    hz1 = hz0 + jnp.uint32(_P3)
                    msk = jnp.uint32(size - 1)
                for corner in range(8):
                    dx, dy, dz = corner & 1, (corner >> 1) & 1, (corner >> 2) & 1
                    if dense:
                        idx = bidx + (dx + dy * r1 + dz * r1 * r1)
                    else:
                        h = ((hx1 if dx else hx0) ^ (hy1 if dy else hy0)
                             ^ (hz1 if dz else hz0))
                        idx = ((h & msk) + jnp.uint32(off)).astype(jnp.int32)
                    w = wxy[dy * 2 + dx] * (fz if dz else wz0)
                    e = idx + idx
                    idx_v[2 * corner, pl.ds(o, _L)] = e
                    idx_v[2 * corner + 1, pl.ds(o, _L)] = e + 1
                    w_v[pl.ds(corner * _C + o, _L)] = w
                return cc

            lax.fori_loop(0, _G, grp_a, 0)

            copies = [pltpu.async_copy(emb_hbm.at[idx_v.at[k]],
                                       rows_v.at[k], sem)
                      for k in range(16)]
            for cp in copies:
                cp.wait()

            iota32 = iota * _OUT_D

            def grp_c(g, cc, l=l):
                o = g * _L
                acc0 = jnp.zeros((_L,), jnp.float32)
                acc1 = jnp.zeros((_L,), jnp.float32)
                for corner in range(8):
                    w = w_v[pl.ds(corner * _C + o, _L)]
                    v0 = rows_v[2 * corner, pl.ds(o, _L)]
                    v1 = rows_v[2 * corner + 1, pl.ds(o, _L)]
                    acc0 = acc0 + w * v0
                    acc1 = acc1 + w * v1
                ovec = iota32 + (o * _OUT_D + 2 * l)
                plsc.store_scatter(out_v, [ovec], acc0)
                plsc.store_scatter(out_v, [ovec + 1], acc1)
                return cc

            lax.fori_loop(0, _G, grp_c, 0)
        pltpu.sync_copy(out_v, out_hbm.at[pl.ds(cbase * _OUT_D, _C * _OUT_D)])
        return carry

    lax.fori_loop(0, _CHUNKS, chunk_body, 0)


def kernel(xyz, embeddings, min_xyz, max_xyz):
    xt = jnp.transpose(xyz).reshape(-1)
    embf = embeddings.reshape(-1)
    inv = 1.0 / (max_xyz - min_xyz)
    mn3 = jnp.broadcast_to(min_xyz[:, None], (3, _L))
    inv3 = jnp.broadcast_to(inv[:, None], (3, _L))
    return _encode_sc(xt, embf, mn3, inv3).reshape(_B, _OUT_D)


# speedup vs baseline: 1.1177x; 1.1177x over previous
import functools

import jax
import jax.numpy as jnp
from jax import lax
from jax.experimental import pallas as pl
from jax.experimental.pallas import tpu as pltpu
from jax.experimental.pallas import tpu_sc as plsc

_N_LEVELS = 16
_N_FEAT = 2
_OFFSETS = [0, 4913, 14174, 31750, 67687, 136608, 269259, 543884, 1068172,
            1592460, 2116748, 2641036, 3165324, 3689612, 4213900, 4738188,
            5262476]
_RES = [16, 20, 25, 32, 40, 50, 64, 80, 101, 128, 161, 203, 256, 322, 406, 512]
_P2 = 2654435761
_P3 = 805459861

_B = 131072
_NC, _NS, _L = 2, 16, 16
_NW = _NC * _NS
_BPW = _B // _NW
_C = 128
_CHUNKS = _BPW // _C
_G = _C // _L
_OUT_D = _N_LEVELS * _N_FEAT

_N_CACHED = 3
_TAB_ROWS = _OFFSETS[_N_CACHED]
_TAB_ELEMS = ((_TAB_ROWS * _N_FEAT + 7) // 8) * 8

_mesh = plsc.VectorSubcoreMesh(core_axis_name="c", subcore_axis_name="s")


@functools.partial(
    pl.kernel,
    mesh=_mesh,
    out_type=jax.ShapeDtypeStruct((_B * _OUT_D,), jnp.float32),
    scratch_types=[
        pltpu.VMEM((3, _L), jnp.float32),
        pltpu.VMEM((3, _L), jnp.float32),
        pltpu.VMEM((3 * _C,), jnp.float32),
        pltpu.VMEM((_TAB_ELEMS,), jnp.float32),
        pltpu.VMEM((2, 16, _C), jnp.int32),
        pltpu.VMEM((8, _C), jnp.int32),
        pltpu.VMEM((2, 8, _C), jnp.float32),
        pltpu.VMEM((8, _C), jnp.float32),
        pltpu.VMEM((2, 16, _C), jnp.float32),
        pltpu.VMEM((_C * _OUT_D,), jnp.float32),
        pltpu.SemaphoreType.DMA,
        pltpu.SemaphoreType.DMA,
    ],
    compiler_params=pltpu.CompilerParams(needs_layout_passes=False),
)
def _encode_sc(xt_hbm, emb_hbm, mn_hbm, inv_hbm, out_hbm,
               mn_v, inv_v, xyz_v, tab_v, idx_v, idxc_v, w_v, wc_v, rows_v,
               out_v, sem0, sem1):
    wid = lax.axis_index("s") * _NC + lax.axis_index("c")
    base = wid * _BPW
    pltpu.sync_copy(mn_hbm, mn_v)
    pltpu.sync_copy(inv_hbm, inv_v)
    pltpu.sync_copy(emb_hbm.at[pl.ds(0, _TAB_ELEMS)], tab_v)
    iota = lax.iota(jnp.int32, _L)
    iota32 = iota * _OUT_D
    sems = (sem0, sem1)

    def phase_a(l, buf, cached):
        res = _RES[l]
        size = _OFFSETS[l + 1] - _OFFSETS[l]
        off = _OFFSETS[l]
        dense = (res + 1) ** 3 <= size
        r1 = res + 1

        def grp_a(g, cc):
            o = g * _L
            ps, fs = [], []
            for a in range(3):
                xa = xyz_v[pl.ds(a * _C + o, _L)]
                xn = jnp.clip((xa - mn_v[a]) * inv_v[a], 0.0, 1.0)
                scl = xn * jnp.float32(res)
                p = jnp.minimum(scl.astype(jnp.int32), res - 1)
                ps.append(p)
                fs.append(scl - p.astype(jnp.float32))
            px, py, pz = ps
            fx, fy, fz = fs
            wx0 = 1.0 - fx
            wy0 = 1.0 - fy
            wz0 = 1.0 - fz
            wxy = (wx0 * wy0, fx * wy0, wx0 * fy, fx * fy)
            if not dense:
                hx0 = px.astype(jnp.uint32)
                hx1 = hx0 + jnp.uint32(1)
                hy0 = py.astype(jnp.uint32) * jnp.uint32(_P2)
                hy1 = hy0 + jnp.uint32(_P2)
                hz0 = pz.astype(jnp.uint32) * jnp.uint32(_P3)
                hz1 = hz0 + jnp.uint32(_P3)
                msk = jnp.uint32(size - 1)
            else:
                bidx = px + py * r1 + pz * (r1 * r1) + off
            for corner in range(8):
                dx, dy, dz = corner & 1, (corner >> 1) & 1, (corner >> 2) & 1
                if dense:
                    idx = bidx + (dx + dy * r1 + dz * r1 * r1)
                else:
                    h = ((hx1 if dx else hx0) ^ (hy1 if dy else hy0)
                         ^ (hz1 if dz else hz0))
                    idx = ((h & msk) + jnp.uint32(off)).astype(jnp.int32)
                w = wxy[dy * 2 + dx] * (fz if dz else wz0)
                if cached:
                    idxc_v[corner, pl.ds(o, _L)] = idx + idx
                    wc_v[corner, pl.ds(o, _L)] = w
                else:
                    e = idx + idx
                    idx_v[buf, 2 * corner, pl.ds(o, _L)] = e
                    idx_v[buf, 2 * corner + 1, pl.ds(o, _L)] = e + 1
                    w_v[buf, corner, pl.ds(o, _L)] = w
            return cc

        lax.fori_loop(0, _G, grp_a, 0)

    def fire(buf):
        return [pltpu.async_copy(emb_hbm.at[idx_v.at[buf, k]],
                                 rows_v.at[buf, k], sems[buf])
                for k in range(16)]

    def phase_c_dma(l, buf):
        def grp_c(g, cc):
            o = g * _L
            acc0 = jnp.zeros((_L,), jnp.float32)
            acc1 = jnp.zeros((_L,), jnp.float32)
            for corner in range(8):
                w = w_v[buf, corner, pl.ds(o, _L)]
                v0 = rows_v[buf, 2 * corner, pl.ds(o, _L)]
                v1 = rows_v[buf, 2 * corner + 1, pl.ds(o, _L)]
                acc0 = acc0 + w * v0
                acc1 = acc1 + w * v1
            ovec = iota32 + (o * _OUT_D + 2 * l)
            plsc.store_scatter(out_v, [ovec], acc0)
            plsc.store_scatter(out_v, [ovec + 1], acc1)
            return cc

        lax.fori_loop(0, _G, grp_c, 0)

    def phase_c_cached(l):
        def grp_c(g, cc):
            o = g * _L
            acc0 = jnp.zeros((_L,), jnp.float32)
            acc1 = jnp.zeros((_L,), jnp.float32)
            for corner in range(8):
                w = wc_v[corner, pl.ds(o, _L)]
                evec = idxc_v[corner, pl.ds(o, _L)]
                v0 = plsc.load_gather(tab_v, [evec])
                v1 = plsc.load_gather(tab_v, [evec + 1])
                acc0 = acc0 + w * v0
                acc1 = acc1 + w * v1
            ovec = iota32 + (o * _OUT_D + 2 * l)
            plsc.store_scatter(out_v, [ovec], acc0)
            plsc.store_scatter(out_v, [ovec + 1], acc1)
            return cc

        lax.fori_loop(0, _G, grp_c, 0)

    def chunk_body(ci, carry):
        cbase = base + ci * _C
        for a in range(3):
            pltpu.sync_copy(xt_hbm.at[pl.ds(a * _B + cbase, _C)],
                            xyz_v.at[pl.ds(a * _C, _C)])
        pend = None
        ndma = 0
        order = [3, 0, 4, 1, 5, 2] + list(range(6, _N_LEVELS))
        for l in order:
            cached = l < _N_CACHED
            buf = ndma % 2
            phase_a(l, buf, cached)
            if cached:
                phase_c_cached(l)
            else:
                copies = fire(buf)
                ndma += 1
                if pend is not None:
                    pl_, pb_, pc_ = pend
                    for cp in pc_:
                        cp.wait()
                    phase_c_dma(pl_, pb_)
                pend = (l, buf, copies)
        pl_, pb_, pc_ = pend
        for cp in pc_:
            cp.wait()
        phase_c_dma(pl_, pb_)
        pltpu.sync_copy(out_v, out_hbm.at[pl.ds(cbase * _OUT_D, _C * _OUT_D)])
        return carry

    lax.fori_loop(0, _CHUNKS, chunk_body, 0)


def kernel(xyz, embeddings, min_xyz, max_xyz):
    xt = jnp.transpose(xyz).reshape(-1)
    embf = embeddings.reshape(-1)
    inv = 1.0 / (max_xyz - min_xyz)
    mn3 = jnp.broadcast_to(min_xyz[:, None], (3, _L))
    inv3 = jnp.broadcast_to(inv[:, None], (3, _L))
    return _encode_sc(xt, embf, mn3, inv3).reshape(_B, _OUT_D)
